# Initial kernel scaffold; baseline (speedup 1.0000x reference)
import functools

import jax
import jax.numpy as jnp
from jax import lax
from jax.experimental import pallas as pl
from jax.experimental.pallas import tpu as pltpu
from jax.experimental.pallas import tpu_sc as plsc

_FEATS = [
    (0, 8),
    (8, 8),
    (16, 4),
    (20, 4),
    (24, 8),
    (32, 8),
    (40, 16),
    (56, 8),
    (64, 8),
    (72, 8),
]
_B = 16384
_D = 80
_NW = 32
_BPW = _B // _NW
_CHUNK = 128
_NCH = _BPW // _CHUNK


def _body(*refs):
    idx_hbm = refs[0:10]
    tables = refs[10:20]
    out_hbm = refs[20]
    idx_v = refs[21]
    stage = refs[22]
    sem = refs[23]

    wid = lax.axis_index("s") * 2 + lax.axis_index("c")
    for f in range(10):
        pltpu.sync_copy(idx_hbm[f].at[wid], idx_v.at[f])
    for c in range(_NCH):
        copies = []
        for f, (col, d) in enumerate(_FEATS):
            copies.append(pltpu.async_copy(
                tables[f].at[idx_v.at[f, c]],
                stage.at[:, pl.ds(col, d)],
                sem,
            ))
        for cp in copies:
            cp.wait()
        base = wid * _BPW + c * _CHUNK
        pltpu.sync_copy(stage, out_hbm.at[pl.ds(base, _CHUNK), :])


@jax.jit
def kernel(city_idx, W_city, state_idx, W_state, store_type_idx, W_store_type,
           cluster_idx, W_cluster, store_nbr_idx, W_store_nbr, family_idx,
           W_family, store_family_interaction_idx, W_store_family_interaction,
           onpromo_promo_sum7_interaction_idx, W_onpromo_promo_sum7_interaction,
           onpromo_state_interaction_idx, W_onpromo_state_interaction,
           promo_sum7_state_interaction_idx, W_promo_sum7_state_interaction):
    idxs = [city_idx, state_idx, store_type_idx, cluster_idx, store_nbr_idx,
            family_idx, store_family_interaction_idx,
            onpromo_promo_sum7_interaction_idx, onpromo_state_interaction_idx,
            promo_sum7_state_interaction_idx]
    tables = [W_city, W_state, W_store_type, W_cluster, W_store_nbr, W_family,
              W_store_family_interaction, W_onpromo_promo_sum7_interaction,
              W_onpromo_state_interaction, W_promo_sum7_state_interaction]
    idxs = [i.reshape(_NW, _NCH, _CHUNK) for i in idxs]

    mesh = plsc.VectorSubcoreMesh(core_axis_name="c", subcore_axis_name="s")
    run = functools.partial(
        pl.kernel,
        mesh=mesh,
        out_type=jax.ShapeDtypeStruct((_B, _D), jnp.float32),
        scratch_types=[
            pltpu.VMEM((10, _NCH, _CHUNK), jnp.int32),
            pltpu.VMEM((_CHUNK, _D), jnp.float32),
            pltpu.SemaphoreType.DMA,
        ],
    )(_body)
    return run(*idxs, *tables)


# speedup vs baseline: 1.3471x; 1.3471x over previous
import functools

import jax
import jax.numpy as jnp
from jax import lax
from jax.experimental import pallas as pl
from jax.experimental.pallas import tpu as pltpu
from jax.experimental.pallas import tpu_sc as plsc

_FEATS = [
    (0, 8),
    (8, 8),
    (16, 4),
    (20, 4),
    (24, 8),
    (32, 8),
    (40, 16),
    (56, 8),
    (64, 8),
    (72, 8),
]
_B = 16384
_D = 80
_NW = 32
_BPW = _B // _NW
_CHUNK = 128
_NCH = _BPW // _CHUNK


def _body(*refs):
    idx_hbm = refs[0:10]
    tables = refs[10:20]
    outs = refs[20:30]
    idx_v = refs[30]
    bufs = refs[31:41]
    sem = refs[41]

    wid = lax.axis_index("s") * 2 + lax.axis_index("c")
    for f in range(10):
        pltpu.sync_copy(idx_hbm[f].at[wid], idx_v.at[f])
    for c in range(_NCH):
        base = wid * _BPW + c * _CHUNK
        copies = []
        for f in range(10):
            copies.append(pltpu.async_copy(
                tables[f].at[idx_v.at[f, c]], bufs[f], sem,
            ))
        for cp in copies:
            cp.wait()
        for f in range(10):
            pltpu.sync_copy(bufs[f], outs[f].at[pl.ds(base, _CHUNK), :])


@jax.jit
def kernel(city_idx, W_city, state_idx, W_state, store_type_idx, W_store_type,
           cluster_idx, W_cluster, store_nbr_idx, W_store_nbr, family_idx,
           W_family, store_family_interaction_idx, W_store_family_interaction,
           onpromo_promo_sum7_interaction_idx, W_onpromo_promo_sum7_interaction,
           onpromo_state_interaction_idx, W_onpromo_state_interaction,
           promo_sum7_state_interaction_idx, W_promo_sum7_state_interaction):
    idxs = [city_idx, state_idx, store_type_idx, cluster_idx, store_nbr_idx,
            family_idx, store_family_interaction_idx,
            onpromo_promo_sum7_interaction_idx, onpromo_state_interaction_idx,
            promo_sum7_state_interaction_idx]
    tables = [W_city, W_state, W_store_type, W_cluster, W_store_nbr, W_family,
              W_store_family_interaction, W_onpromo_promo_sum7_interaction,
              W_onpromo_state_interaction, W_promo_sum7_state_interaction]
    idxs = [i.reshape(_NW, _NCH, _CHUNK) for i in idxs]
    gws = [max(d, 8) for _, d in _FEATS]
    tables = [t if t.shape[1] >= 8 else jnp.pad(t, ((0, 0), (0, 8 - t.shape[1])))
              for t in tables]

    mesh = plsc.VectorSubcoreMesh(core_axis_name="c", subcore_axis_name="s")
    run = functools.partial(
        pl.kernel,
        mesh=mesh,
        out_type=[jax.ShapeDtypeStruct((_B, gw), jnp.float32) for gw in gws],
        scratch_types=[
            pltpu.VMEM((10, _NCH, _CHUNK), jnp.int32),
            *[pltpu.VMEM((_CHUNK, gw), jnp.float32) for gw in gws],
            pltpu.SemaphoreType.DMA,
        ],
        compiler_params=pltpu.CompilerParams(use_tc_tiling_on_sc=False),
    )(_body)
    outs = run(*idxs, *tables)
    outs = [o[:, :d] for o, (_, d) in zip(outs, _FEATS)]
    return jnp.concatenate(outs, axis=-1)


# speedup vs baseline: 2.1652x; 1.6073x over previous
import functools

import jax
import jax.numpy as jnp
from jax import lax
from jax.experimental import pallas as pl
from jax.experimental.pallas import tpu as pltpu
from jax.experimental.pallas import tpu_sc as plsc

_B = 16384
_D = 80
_NW = 32
_BPW = _B // _NW
_CHUNK = 128
_NCH = _BPW // _CHUNK

_SMALL = [
    (0, 0, 8, 0),
    (1, 8, 8, 23),
    (2, 16, 4, 40),
    (3, 20, 4, 46),
    (4, 24, 8, 64),
    (5, 32, 8, 119),
]
_SMALL_ROWS = 153
_BIG = [
    (6, 40, 16),
    (7, 56, 8),
    (8, 64, 8),
    (9, 72, 8),
]


def _body(*refs):
    idx_hbm = refs[0:10]
    small_hbm = refs[10]
    bigtabs = refs[11:15]
    out_hbm = refs[15]
    idx_v = refs[16]
    small_v = refs[17]
    bufs = refs[18:22]
    stage = refs[22]
    sem = refs[23]

    wid = lax.axis_index("s") * 2 + lax.axis_index("c")
    for f in range(10):
        pltpu.sync_copy(idx_hbm[f].at[wid], idx_v.at[f])
    pltpu.sync_copy(small_hbm, small_v)

    iota = lax.iota(jnp.int32, 16)
    rowpat = iota * _D
    pat8 = (iota >> 3) * _D + (iota & 7)
    prow16 = (iota >> 3)
    pcol8 = iota & 7

    for c in range(_NCH):
        base = wid * _BPW + c * _CHUNK
        copies = [
            pltpu.async_copy(bigtabs[k].at[idx_v.at[s, c]], bufs[k], sem)
            for k, (s, col, w) in enumerate(_BIG)
        ]

        def small_group(g, carry):
            rb = g * 16 * _D
            for s, col, w, off in _SMALL:
                idxv = idx_v[s, c, pl.ds(g * 16, 16)]
                rowbase = (idxv << 3) + (off * 8)
                for k in range(w):
                    vals = plsc.load_gather(small_v, [rowbase + k])
                    plsc.store_scatter(stage, [rowpat + (rb + col + k)], vals)
            return carry

        lax.fori_loop(0, _CHUNK // 16, small_group, 0, unroll=False)

        for cp in copies:
            cp.wait()

        def row16(j, carry):
            v = bufs[0][j, :]
            plsc.store_scatter(stage, [iota + (j * _D + 40)], v)
            return carry

        lax.fori_loop(0, _CHUNK, row16, 0, unroll=4)

        def pair8(j, carry):
            rows = prow16 + (j * 2)
            dbase = pat8 + (j * 2 * _D)
            for k, (s, col, w) in enumerate(_BIG[1:], start=1):
                v = plsc.load_gather(bufs[k], [rows, pcol8])
                plsc.store_scatter(stage, [dbase + col], v)
            return carry

        lax.fori_loop(0, _CHUNK // 2, pair8, 0, unroll=2)

        pltpu.sync_copy(stage, out_hbm.at[pl.ds(base * _D, _CHUNK * _D)])


@jax.jit
def kernel(city_idx, W_city, state_idx, W_state, store_type_idx, W_store_type,
           cluster_idx, W_cluster, store_nbr_idx, W_store_nbr, family_idx,
           W_family, store_family_interaction_idx, W_store_family_interaction,
           onpromo_promo_sum7_interaction_idx, W_onpromo_promo_sum7_interaction,
           onpromo_state_interaction_idx, W_onpromo_state_interaction,
           promo_sum7_state_interaction_idx, W_promo_sum7_state_interaction):
    idxs = [city_idx, state_idx, store_type_idx, cluster_idx, store_nbr_idx,
            family_idx, store_family_interaction_idx,
            onpromo_promo_sum7_interaction_idx, onpromo_state_interaction_idx,
            promo_sum7_state_interaction_idx]
    idxs = [i.reshape(_NW, _NCH, _CHUNK) for i in idxs]
    smalls = [W_city, W_state, W_store_type, W_cluster, W_store_nbr, W_family]
    small_tab = jnp.concatenate(
        [jnp.pad(t, ((0, 0), (0, 8 - t.shape[1]))) for t in smalls], axis=0
    ).reshape(-1)
    bigtabs = [W_store_family_interaction, W_onpromo_promo_sum7_interaction,
               W_onpromo_state_interaction, W_promo_sum7_state_interaction]

    mesh = plsc.VectorSubcoreMesh(core_axis_name="c", subcore_axis_name="s")
    run = functools.partial(
        pl.kernel,
        mesh=mesh,
        out_type=jax.ShapeDtypeStruct((_B * _D,), jnp.float32),
        scratch_types=[
            pltpu.VMEM((10, _NCH, _CHUNK), jnp.int32),
            pltpu.VMEM((_SMALL_ROWS * 8,), jnp.float32),
            pltpu.VMEM((_CHUNK, 16), jnp.float32),
            pltpu.VMEM((_CHUNK, 8), jnp.float32),
            pltpu.VMEM((_CHUNK, 8), jnp.float32),
            pltpu.VMEM((_CHUNK, 8), jnp.float32),
            pltpu.VMEM((_CHUNK * _D,), jnp.float32),
            pltpu.SemaphoreType.DMA,
        ],
        compiler_params=pltpu.CompilerParams(use_tc_tiling_on_sc=False, needs_layout_passes=False),
    )(_body)
    out = run(*idxs, small_tab, *bigtabs)
    return out.reshape(_B, _D)
